# baseline (speedup 1.0000x reference)
import functools

import jax
import jax.numpy as jnp
from jax import lax
from jax.experimental import pallas as pl
from jax.experimental.pallas import tpu as pltpu
from jax.experimental.pallas import tpu_sc as plsc

SEQ = 2048
D_MODEL = 768
N_EXPERTS = 8
TOP_K = 2
D_FF = 4 * D_MODEL

NPAIR = SEQ * TOP_K
TM = 256
NP_ROWS = NPAIR + N_EXPERTS * TM
NT = NP_ROWS // TM
FB = 768
NJ = D_FF // FB


def _gating_body(x_ref, wg_ref, bg_ref, idx_ref, wts_ref):
    x = x_ref[...]
    logits = jnp.dot(x, wg_ref[...], preferred_element_type=jnp.float32)
    logits = logits + bg_ref[...]
    iota = lax.broadcasted_iota(jnp.int32, logits.shape, 1)
    m1 = jnp.max(logits, axis=1, keepdims=True)
    i1 = jnp.min(jnp.where(logits >= m1, iota, N_EXPERTS), axis=1, keepdims=True)
    lmask = jnp.where(iota == i1, -jnp.inf, logits)
    m2 = jnp.max(lmask, axis=1, keepdims=True)
    i2 = jnp.min(jnp.where(lmask >= m2, iota, N_EXPERTS), axis=1, keepdims=True)
    z = jnp.sum(jnp.exp(logits - m1), axis=1, keepdims=True)
    w1 = 1.0 / z
    w2 = jnp.exp(m2 - m1) / z
    idx_ref[...] = jnp.concatenate([i1, i2], axis=1)
    wts_ref[...] = jnp.concatenate([w1, w2], axis=1)


def _gating(xt, Wg, bg):
    return pl.pallas_call(
        _gating_body,
        out_shape=(
            jax.ShapeDtypeStruct((SEQ, TOP_K), jnp.int32),
            jax.ShapeDtypeStruct((SEQ, TOP_K), jnp.float32),
        ),
    )(xt, Wg, bg.reshape(1, N_EXPERTS))


def _sc_gather(table, idx):
    info = plsc.get_sparse_core_info()
    _NC, _NS = info.num_cores, info.num_subcores
    _NW = _NC * _NS
    V, D = table.shape
    B = idx.shape[0]
    b_per_w = B // _NW
    chunk = b_per_w
    while chunk * D * 4 > 450_000:
        chunk //= 2
    n_chunks = b_per_w // chunk
    mesh = plsc.VectorSubcoreMesh(core_axis_name="c", subcore_axis_name="s")

    @functools.partial(
        pl.kernel,
        mesh=mesh,
        out_type=jax.ShapeDtypeStruct((B, D), jnp.float32),
        scratch_types=[
            pltpu.VMEM((chunk,), jnp.int32),
            pltpu.VMEM((chunk, D), jnp.float32),
            pltpu.SemaphoreType.DMA,
        ],
    )
    def k(table_hbm, idx_hbm, out_hbm, idx_v, rows_v, sem):
        wid = lax.axis_index("s") * _NC + lax.axis_index("c")
        for c in range(n_chunks):
            base = wid * b_per_w + c * chunk
            pltpu.sync_copy(idx_hbm.at[pl.ds(base, chunk)], idx_v)
            pltpu.async_copy(table_hbm.at[idx_v], rows_v, sem).wait()
            pltpu.sync_copy(rows_v, out_hbm.at[pl.ds(base, chunk)])

    return k(table, idx)


def _gmm_body(te_ref, xs_ref, w1_ref, b1_ref, w2_ref, b2_ref, ws_ref, y_ref,
              acc_ref):
    j = pl.program_id(0)
    t = pl.program_id(1)
    x = xs_ref[...]
    w1 = w1_ref[0].astype(jnp.bfloat16)
    h = jnp.dot(x, w1, preferred_element_type=jnp.float32)
    h = jnp.maximum(h + b1_ref[0, 0][None, :], 0.0).astype(jnp.bfloat16)
    w2 = w2_ref[0].astype(jnp.bfloat16)
    yp = jnp.dot(h, w2, preferred_element_type=jnp.float32)
    ws = ws_ref[0, 0][:, None]
    sl = pl.ds(t * TM, TM)

    @pl.when(j == 0)
    def _():
        acc_ref[sl, :] = ws * (yp + b2_ref[0, 0][None, :])

    @pl.when(j > 0)
    def _():
        acc_ref[sl, :] += ws * yp

    @pl.when(j == NJ - 1)
    def _():
        y_ref[...] = acc_ref[sl, :]


def _gmm(tile_expert, xs, W1, b1, W2, b2, ws_sorted):
    grid_spec = pltpu.PrefetchScalarGridSpec(
        num_scalar_prefetch=1,
        grid=(NJ, NT),
        in_specs=[
            pl.BlockSpec((TM, D_MODEL), lambda j, t, te: (t, 0)),
            pl.BlockSpec((1, D_MODEL, FB), lambda j, t, te: (te[t], 0, j)),
            pl.BlockSpec((1, 1, FB), lambda j, t, te: (te[t], 0, j)),
            pl.BlockSpec((1, FB, D_MODEL), lambda j, t, te: (te[t], j, 0)),
            pl.BlockSpec((1, 1, D_MODEL), lambda j, t, te: (te[t], 0, 0)),
            pl.BlockSpec((1, 1, TM), lambda j, t, te: (t, 0, 0)),
        ],
        out_specs=pl.BlockSpec((TM, D_MODEL), lambda j, t, te: (t, 0)),
        scratch_shapes=[pltpu.VMEM((NP_ROWS, D_MODEL), jnp.float32)],
    )
    return pl.pallas_call(
        _gmm_body,
        grid_spec=grid_spec,
        out_shape=jax.ShapeDtypeStruct((NP_ROWS, D_MODEL), jnp.float32),
    )(
        tile_expert,
        xs.astype(jnp.bfloat16),
        W1,
        b1.reshape(N_EXPERTS, 1, D_FF),
        W2,
        b2.reshape(N_EXPERTS, 1, D_MODEL),
        ws_sorted.reshape(NT, 1, TM),
    )


def _sum_body(a_ref, o_ref):
    a = a_ref[...]
    o_ref[...] = a[:, :D_MODEL] + a[:, D_MODEL:]


def _combine_sum(ysel):
    return pl.pallas_call(
        _sum_body,
        grid=(SEQ // 256,),
        in_specs=[pl.BlockSpec((256, 2 * D_MODEL), lambda t: (t, 0))],
        out_specs=pl.BlockSpec((256, D_MODEL), lambda t: (t, 0)),
        out_shape=jax.ShapeDtypeStruct((SEQ, D_MODEL), jnp.float32),
    )(ysel.reshape(SEQ, 2 * D_MODEL))


def kernel(x, Wg, bg, W1, b1, W2, b2):
    B, S, D = x.shape
    xt = x.reshape(S, D)

    idx, wts = _gating(xt, Wg, bg)

    e_flat = idx.reshape(NPAIR)
    oh = (e_flat[:, None] == jnp.arange(N_EXPERTS, dtype=jnp.int32)[None, :])
    csum = jnp.cumsum(oh.astype(jnp.int32), axis=0)
    counts = csum[-1]
    cpad = ((counts + TM - 1) // TM) * TM
    ends = jnp.cumsum(cpad)
    starts = ends - cpad
    rank = jnp.take_along_axis(csum, e_flat[:, None], axis=1)[:, 0] - 1
    dest = (starts[e_flat] + rank).astype(jnp.int32)
    tok_sorted = jnp.zeros((NP_ROWS,), jnp.int32).at[dest].set(
        jnp.arange(NPAIR, dtype=jnp.int32) // TOP_K)
    ws_sorted = jnp.zeros((NP_ROWS,), jnp.float32).at[dest].set(wts.reshape(NPAIR))
    tile_expert = jnp.minimum(
        jnp.searchsorted(ends, jnp.arange(NT, dtype=jnp.int32) * TM, side="right"),
        N_EXPERTS - 1,
    ).astype(jnp.int32)

    xs = _sc_gather(xt, tok_sorted)
    y = _gmm(tile_expert, xs, W1, b1, W2, b2, ws_sorted)
    ysel = _sc_gather(y, dest)
    out = _combine_sum(ysel)
    return out.reshape(B, S, D)


# speedup vs baseline: 1.1372x; 1.1372x over previous
import jax
import jax.numpy as jnp
from jax import lax
from jax.experimental import pallas as pl
from jax.experimental.pallas import tpu as pltpu

SEQ = 2048
D_MODEL = 768
N_EXPERTS = 8
TOP_K = 2
D_FF = 4 * D_MODEL

NPAIR = SEQ * TOP_K
TM = 128
NP_ROWS = NPAIR + N_EXPERTS * TM
NT = NP_ROWS // TM
FB = 768
NJ = D_FF // FB


def _gating_body(x_ref, wg_ref, bg_ref, idx_ref, wts_ref):
    x = x_ref[...]
    logits = jnp.dot(x, wg_ref[...], preferred_element_type=jnp.float32)
    logits = logits + bg_ref[...]
    iota = lax.broadcasted_iota(jnp.int32, logits.shape, 1)
    m1 = jnp.max(logits, axis=1, keepdims=True)
    i1 = jnp.min(jnp.where(logits >= m1, iota, N_EXPERTS), axis=1, keepdims=True)
    lmask = jnp.where(iota == i1, -jnp.inf, logits)
    m2 = jnp.max(lmask, axis=1, keepdims=True)
    i2 = jnp.min(jnp.where(lmask >= m2, iota, N_EXPERTS), axis=1, keepdims=True)
    z = jnp.sum(jnp.exp(logits - m1), axis=1, keepdims=True)
    w1 = 1.0 / z
    w2 = jnp.exp(m2 - m1) / z
    idx_ref[...] = jnp.concatenate([i1, i2], axis=1)
    wts_ref[...] = jnp.concatenate([w1, w2], axis=1)


def _gating(xt, Wg, bg):
    return pl.pallas_call(
        _gating_body,
        out_shape=(
            jax.ShapeDtypeStruct((SEQ, TOP_K), jnp.int32),
            jax.ShapeDtypeStruct((SEQ, TOP_K), jnp.float32),
        ),
    )(xt, Wg, bg.reshape(1, N_EXPERTS))


def _moe_body(meta_ref, x_ref, w1_ref, b1_ref, w2_ref, b2_ref, tok_ref,
              ws_ref, out_ref, xs_ref, acc_ref, oacc_ref):
    j = pl.program_id(0)
    t = pl.program_id(1)
    sl = pl.ds(t * TM, TM)
    valid = meta_ref[NT + t] == 1

    @pl.when(valid)
    def _():
        tok = tok_ref[0, 0]

        @pl.when(j == 0)
        def _():
            iota_n = lax.broadcasted_iota(jnp.int32, (TM, SEQ), 1)
            p = jnp.where(iota_n == tok[:, None], 1.0, 0.0).astype(jnp.bfloat16)
            xs = jnp.dot(p, x_ref[...], preferred_element_type=jnp.float32)
            xs_ref[sl, :] = xs.astype(jnp.bfloat16)

        xv = xs_ref[sl, :]
        w1 = w1_ref[0].astype(jnp.bfloat16)
        h = jnp.dot(xv, w1, preferred_element_type=jnp.float32)
        h = jnp.maximum(h + b1_ref[0, 0][None, :], 0.0).astype(jnp.bfloat16)
        w2 = w2_ref[0].astype(jnp.bfloat16)
        yp = jnp.dot(h, w2, preferred_element_type=jnp.float32)

        @pl.when(j == 0)
        def _():
            acc_ref[sl, :] = yp

        @pl.when(j > 0)
        def _():
            acc_ref[sl, :] += yp

        @pl.when(j == NJ - 1)
        def _():
            y = acc_ref[sl, :] + b2_ref[0, 0][None, :]
            ws = ws_ref[0, 0]
            iota_t = lax.broadcasted_iota(jnp.int32, (SEQ, TM), 0)
            gt = jnp.where(iota_t == tok[None, :], ws[None, :],
                           0.0).astype(jnp.bfloat16)
            contrib = jnp.dot(gt, y.astype(jnp.bfloat16),
                              preferred_element_type=jnp.float32)

            @pl.when(t == 0)
            def _():
                oacc_ref[...] = contrib

            @pl.when(t > 0)
            def _():
                oacc_ref[...] += contrib

    @pl.when(jnp.logical_and(j == NJ - 1, t == NT - 1))
    def _():
        out_ref[...] = oacc_ref[...]


def _moe(meta, xb, W1, b1, W2, b2, tok3, ws3):
    grid_spec = pltpu.PrefetchScalarGridSpec(
        num_scalar_prefetch=1,
        grid=(NJ, NT),
        in_specs=[
            pl.BlockSpec((SEQ, D_MODEL), lambda j, t, m: (0, 0)),
            pl.BlockSpec((1, D_MODEL, FB), lambda j, t, m: (m[t], 0, j)),
            pl.BlockSpec((1, 1, FB), lambda j, t, m: (m[t], 0, j)),
            pl.BlockSpec((1, FB, D_MODEL), lambda j, t, m: (m[t], j, 0)),
            pl.BlockSpec((1, 1, D_MODEL), lambda j, t, m: (m[t], 0, 0)),
            pl.BlockSpec((1, 1, TM), lambda j, t, m: (t, 0, 0)),
            pl.BlockSpec((1, 1, TM), lambda j, t, m: (t, 0, 0)),
        ],
        out_specs=pl.BlockSpec((SEQ, D_MODEL), lambda j, t, m: (0, 0)),
        scratch_shapes=[
            pltpu.VMEM((NP_ROWS, D_MODEL), jnp.bfloat16),
            pltpu.VMEM((NP_ROWS, D_MODEL), jnp.float32),
            pltpu.VMEM((SEQ, D_MODEL), jnp.float32),
        ],
    )
    return pl.pallas_call(
        _moe_body,
        grid_spec=grid_spec,
        out_shape=jax.ShapeDtypeStruct((SEQ, D_MODEL), jnp.float32),
    )(
        meta,
        xb,
        W1,
        b1.reshape(N_EXPERTS, 1, D_FF),
        W2,
        b2.reshape(N_EXPERTS, 1, D_MODEL),
        tok3,
        ws3,
    )


def kernel(x, Wg, bg, W1, b1, W2, b2):
    B, S, D = x.shape
    xt = x.reshape(S, D)

    idx, wts = _gating(xt, Wg, bg)

    e_flat = idx.reshape(NPAIR)
    oh = (e_flat[:, None] == jnp.arange(N_EXPERTS, dtype=jnp.int32)[None, :])
    csum = jnp.cumsum(oh.astype(jnp.int32), axis=0)
    counts = csum[-1]
    cpad = ((counts + TM - 1) // TM) * TM
    ends = jnp.cumsum(cpad)
    starts = ends - cpad
    rank = jnp.take_along_axis(csum, e_flat[:, None], axis=1)[:, 0] - 1
    dest = (starts[e_flat] + rank).astype(jnp.int32)
    tok_sorted = jnp.zeros((NP_ROWS,), jnp.int32).at[dest].set(
        jnp.arange(NPAIR, dtype=jnp.int32) // TOP_K)
    ws_sorted = jnp.zeros((NP_ROWS,), jnp.float32).at[dest].set(wts.reshape(NPAIR))
    tvec = jnp.arange(NT, dtype=jnp.int32) * TM
    tile_expert = jnp.minimum(
        jnp.searchsorted(ends, tvec, side="right"), N_EXPERTS - 1
    ).astype(jnp.int32)
    tile_valid = (tvec < ends[N_EXPERTS - 1]).astype(jnp.int32)
    meta = jnp.concatenate([tile_expert, tile_valid])

    out = _moe(meta, xt.astype(jnp.bfloat16), W1, b1, W2, b2,
               tok_sorted.reshape(NT, 1, TM), ws_sorted.reshape(NT, 1, TM))
    return out.reshape(B, S, D)


# speedup vs baseline: 1.6130x; 1.4184x over previous
import jax
import jax.numpy as jnp
from jax import lax
from jax.experimental import pallas as pl
from jax.experimental.pallas import tpu as pltpu

SEQ = 2048
D_MODEL = 768
N_EXPERTS = 8
TOP_K = 2
D_FF = 4 * D_MODEL

NPAIR = SEQ * TOP_K
TM = 256
NP_ROWS = NPAIR + N_EXPERTS * TM
NT = NP_ROWS // TM
FB = 768
NJ = D_FF // FB


def _gating_body(x_ref, wg_ref, bg_ref, idx_ref, wts_ref):
    x = x_ref[...]
    logits = jnp.dot(x, wg_ref[...], preferred_element_type=jnp.float32)
    logits = logits + bg_ref[...]
    iota = lax.broadcasted_iota(jnp.int32, logits.shape, 1)
    m1 = jnp.max(logits, axis=1, keepdims=True)
    i1 = jnp.min(jnp.where(logits >= m1, iota, N_EXPERTS), axis=1, keepdims=True)
    lmask = jnp.where(iota == i1, -jnp.inf, logits)
    m2 = jnp.max(lmask, axis=1, keepdims=True)
    i2 = jnp.min(jnp.where(lmask >= m2, iota, N_EXPERTS), axis=1, keepdims=True)
    z = jnp.sum(jnp.exp(logits - m1), axis=1, keepdims=True)
    w1 = 1.0 / z
    w2 = jnp.exp(m2 - m1) / z
    idx_ref[...] = jnp.concatenate([i1, i2], axis=1)
    wts_ref[...] = jnp.concatenate([w1, w2], axis=1)


def _gating(xt, Wg, bg):
    return pl.pallas_call(
        _gating_body,
        out_shape=(
            jax.ShapeDtypeStruct((SEQ, TOP_K), jnp.int32),
            jax.ShapeDtypeStruct((SEQ, TOP_K), jnp.float32),
        ),
    )(xt, Wg, bg.reshape(1, N_EXPERTS))


def _moe_body(meta_ref, x_ref, w1_ref, b1_ref, w2_ref, b2_ref, d2_ref,
              wts_ref, out_ref, xs_ref, acc_ref, oacc_ref):
    j = pl.program_id(0)
    t = pl.program_id(1)
    sl = pl.ds(t * TM, TM)
    base = t * TM
    valid = meta_ref[NT + t] == 1

    @pl.when(valid)
    def _():
        d0r = d2_ref[0, :]
        d1r = d2_ref[1, :]

        @pl.when(j == 0)
        def _():
            iota_r = lax.broadcasted_iota(jnp.int32, (TM, SEQ), 0) + base
            hit = jnp.logical_or(d0r[None, :] == iota_r, d1r[None, :] == iota_r)
            p = jnp.where(hit, 1.0, 0.0).astype(jnp.bfloat16)
            xs = jnp.dot(p, x_ref[...], preferred_element_type=jnp.float32)
            xs_ref[sl, :] = xs.astype(jnp.bfloat16)

        xv = xs_ref[sl, :]
        w1 = w1_ref[0].astype(jnp.bfloat16)
        h = jnp.dot(xv, w1, preferred_element_type=jnp.float32)
        h = jnp.maximum(h + b1_ref[0, 0][None, :], 0.0).astype(jnp.bfloat16)
        w2 = w2_ref[0].astype(jnp.bfloat16)
        yp = jnp.dot(h, w2, preferred_element_type=jnp.float32)

        @pl.when(j == 0)
        def _():
            acc_ref[sl, :] = yp

        @pl.when(j > 0)
        def _():
            acc_ref[sl, :] += yp

        @pl.when(j == NJ - 1)
        def _():
            y = acc_ref[sl, :] + b2_ref[0, 0][None, :]
            w = wts_ref[...]
            iota_c = lax.broadcasted_iota(jnp.int32, (SEQ, TM), 1) + base
            gt = (jnp.where(d0r[:, None] == iota_c, w[:, :1], 0.0)
                  + jnp.where(d1r[:, None] == iota_c, w[:, 1:2], 0.0)
                  ).astype(jnp.bfloat16)
            contrib = jnp.dot(gt, y.astype(jnp.bfloat16),
                              preferred_element_type=jnp.float32)

            @pl.when(t == 0)
            def _():
                oacc_ref[...] = contrib

            @pl.when(t > 0)
            def _():
                oacc_ref[...] += contrib

    @pl.when(jnp.logical_and(j == NJ - 1, t == NT - 1))
    def _():
        out_ref[...] = oacc_ref[...]


def _moe(meta, xb, W1, b1, W2, b2, dest2, wts):
    grid_spec = pltpu.PrefetchScalarGridSpec(
        num_scalar_prefetch=1,
        grid=(NJ, NT),
        in_specs=[
            pl.BlockSpec((SEQ, D_MODEL), lambda j, t, m: (0, 0)),
            pl.BlockSpec((1, D_MODEL, FB), lambda j, t, m: (m[t], 0, j)),
            pl.BlockSpec((1, 1, FB), lambda j, t, m: (m[t], 0, j)),
            pl.BlockSpec((1, FB, D_MODEL), lambda j, t, m: (m[t], j, 0)),
            pl.BlockSpec((1, 1, D_MODEL), lambda j, t, m: (m[t], 0, 0)),
            pl.BlockSpec((2, SEQ), lambda j, t, m: (0, 0)),
            pl.BlockSpec((SEQ, TOP_K), lambda j, t, m: (0, 0)),
        ],
        out_specs=pl.BlockSpec((SEQ, D_MODEL), lambda j, t, m: (0, 0)),
        scratch_shapes=[
            pltpu.VMEM((NP_ROWS, D_MODEL), jnp.bfloat16),
            pltpu.VMEM((NP_ROWS, D_MODEL), jnp.float32),
            pltpu.VMEM((SEQ, D_MODEL), jnp.float32),
        ],
    )
    return pl.pallas_call(
        _moe_body,
        grid_spec=grid_spec,
        out_shape=jax.ShapeDtypeStruct((SEQ, D_MODEL), jnp.float32),
    )(
        meta,
        xb,
        W1,
        b1.reshape(N_EXPERTS, 1, D_FF),
        W2,
        b2.reshape(N_EXPERTS, 1, D_MODEL),
        dest2,
        wts,
    )


def kernel(x, Wg, bg, W1, b1, W2, b2):
    B, S, D = x.shape
    xt = x.reshape(S, D)

    idx, wts = _gating(xt, Wg, bg)

    e_flat = idx.T.reshape(NPAIR)
    oh = (e_flat[:, None] == jnp.arange(N_EXPERTS, dtype=jnp.int32)[None, :])
    csum = jnp.cumsum(oh.astype(jnp.int32), axis=0)
    counts = csum[-1]
    cpad = ((counts + TM - 1) // TM) * TM
    ends = jnp.cumsum(cpad)
    starts = ends - cpad
    rank = jnp.take_along_axis(csum, e_flat[:, None], axis=1)[:, 0] - 1
    dest = (starts[e_flat] + rank).astype(jnp.int32)
    tvec = jnp.arange(NT, dtype=jnp.int32) * TM
    tile_expert = jnp.minimum(
        jnp.searchsorted(ends, tvec, side="right"), N_EXPERTS - 1
    ).astype(jnp.int32)
    tile_valid = (tvec < ends[N_EXPERTS - 1]).astype(jnp.int32)
    meta = jnp.concatenate([tile_expert, tile_valid])

    out = _moe(meta, xt.astype(jnp.bfloat16), W1, b1, W2, b2,
               dest.reshape(TOP_K, SEQ), wts)
    return out.reshape(B, S, D)


# speedup vs baseline: 1.7933x; 1.1118x over previous
import jax
import jax.numpy as jnp
from jax import lax
from jax.experimental import pallas as pl
from jax.experimental.pallas import tpu as pltpu

SEQ = 2048
D_MODEL = 768
N_EXPERTS = 8
TOP_K = 2
D_FF = 4 * D_MODEL

NPAIR = SEQ * TOP_K
TM = 256
NP_ROWS = NPAIR + N_EXPERTS * TM
NT = NP_ROWS // TM
FB = 768
NJ = D_FF // FB


def _gate_route_body(x_ref, wg_ref, bg_ref, wts_ref, dest_ref, meta_ref):
    x = x_ref[...]
    logits = jnp.dot(x, wg_ref[...], preferred_element_type=jnp.float32)
    logits = logits + bg_ref[...]
    iota = lax.broadcasted_iota(jnp.int32, logits.shape, 1)
    m1 = jnp.max(logits, axis=1, keepdims=True)
    i1 = jnp.min(jnp.where(logits >= m1, iota, N_EXPERTS), axis=1, keepdims=True)
    lmask = jnp.where(iota == i1, -jnp.inf, logits)
    m2 = jnp.max(lmask, axis=1, keepdims=True)
    i2 = jnp.min(jnp.where(lmask >= m2, iota, N_EXPERTS), axis=1, keepdims=True)
    z = jnp.sum(jnp.exp(logits - m1), axis=1, keepdims=True)
    wts_ref[...] = jnp.concatenate([1.0 / z, jnp.exp(m2 - m1) / z], axis=1)

    iotaE1 = lax.broadcasted_iota(jnp.int32, (SEQ, N_EXPERTS), 1)
    oh = jnp.concatenate([(iotaE1 == i1), (iotaE1 == i2)], axis=0)
    oh = oh.astype(jnp.int32)
    s = oh
    d = 1
    while d < NPAIR:
        s = s + jnp.concatenate(
            [jnp.zeros((d, N_EXPERTS), jnp.int32), s[: NPAIR - d, :]], axis=0)
        d *= 2
    rank = jnp.sum(oh * s, axis=1, keepdims=True)
    counts = s[NPAIR - 1 :, :]
    cpad = ((counts + TM - 1) // TM) * TM
    e = cpad
    d = 1
    while d < N_EXPERTS:
        e = e + jnp.concatenate(
            [jnp.zeros((1, d), jnp.int32), e[:, : N_EXPERTS - d]], axis=1)
        d *= 2
    ends = e
    starts = ends - cpad
    start_pp = jnp.sum(oh * starts, axis=1, keepdims=True)
    dest_ref[...] = start_pp + rank - 1

    tvec = lax.broadcasted_iota(jnp.int32, (NT, N_EXPERTS), 0) * TM
    te = jnp.minimum(jnp.sum((tvec >= ends).astype(jnp.int32), axis=1,
                             keepdims=True), N_EXPERTS - 1)
    valid = (tvec[:, :1] < ends[:, N_EXPERTS - 1 :]).astype(jnp.int32)
    meta_ref[...] = jnp.concatenate([te, valid], axis=0)


def _gate_route(xt, Wg, bg):
    return pl.pallas_call(
        _gate_route_body,
        out_shape=(
            jax.ShapeDtypeStruct((SEQ, TOP_K), jnp.float32),
            jax.ShapeDtypeStruct((NPAIR, 1), jnp.int32),
            jax.ShapeDtypeStruct((2 * NT, 1), jnp.int32),
        ),
    )(xt, Wg, bg.reshape(1, N_EXPERTS))


def _moe_body(meta_ref, x_ref, w1_ref, b1_ref, w2_ref, b2_ref, d2_ref,
              wts_ref, out_ref, xs_ref, acc_ref, oacc_ref):
    j = pl.program_id(0)
    t = pl.program_id(1)
    sl = pl.ds(t * TM, TM)
    base = t * TM
    valid = meta_ref[NT + t] == 1

    @pl.when(valid)
    def _():
        d0r = d2_ref[0, :]
        d1r = d2_ref[1, :]

        @pl.when(j == 0)
        def _():
            iota_r = lax.broadcasted_iota(jnp.int32, (TM, SEQ), 0) + base
            hit = jnp.logical_or(d0r[None, :] == iota_r, d1r[None, :] == iota_r)
            p = jnp.where(hit, 1.0, 0.0).astype(jnp.bfloat16)
            xs = jnp.dot(p, x_ref[...], preferred_element_type=jnp.float32)
            xs_ref[sl, :] = xs.astype(jnp.bfloat16)

        xv = xs_ref[sl, :]
        w1 = w1_ref[0].astype(jnp.bfloat16)
        h = jnp.dot(xv, w1, preferred_element_type=jnp.float32)
        h = jnp.maximum(h + b1_ref[0, 0][None, :], 0.0).astype(jnp.bfloat16)
        w2 = w2_ref[0].astype(jnp.bfloat16)
        yp = jnp.dot(h, w2, preferred_element_type=jnp.float32)

        @pl.when(j == 0)
        def _():
            acc_ref[sl, :] = yp

        @pl.when(j > 0)
        def _():
            acc_ref[sl, :] += yp

        @pl.when(j == NJ - 1)
        def _():
            y = acc_ref[sl, :] + b2_ref[0, 0][None, :]
            w = wts_ref[...]
            iota_c = lax.broadcasted_iota(jnp.int32, (SEQ, TM), 1) + base
            gt = (jnp.where(d0r[:, None] == iota_c, w[:, :1], 0.0)
                  + jnp.where(d1r[:, None] == iota_c, w[:, 1:2], 0.0)
                  ).astype(jnp.bfloat16)
            contrib = jnp.dot(gt, y.astype(jnp.bfloat16),
                              preferred_element_type=jnp.float32)

            @pl.when(t == 0)
            def _():
                oacc_ref[...] = contrib

            @pl.when(t > 0)
            def _():
                oacc_ref[...] += contrib

    @pl.when(jnp.logical_and(j == NJ - 1, t == NT - 1))
    def _():
        out_ref[...] = oacc_ref[...]


def _moe(meta, xb, W1, b1, W2, b2, dest2, wts):
    grid_spec = pltpu.PrefetchScalarGridSpec(
        num_scalar_prefetch=1,
        grid=(NJ, NT),
        in_specs=[
            pl.BlockSpec((SEQ, D_MODEL), lambda j, t, m: (0, 0)),
            pl.BlockSpec((1, D_MODEL, FB), lambda j, t, m: (m[t], 0, j)),
            pl.BlockSpec((1, 1, FB), lambda j, t, m: (m[t], 0, j)),
            pl.BlockSpec((1, FB, D_MODEL), lambda j, t, m: (m[t], j, 0)),
            pl.BlockSpec((1, 1, D_MODEL), lambda j, t, m: (m[t], 0, 0)),
            pl.BlockSpec((2, SEQ), lambda j, t, m: (0, 0)),
            pl.BlockSpec((SEQ, TOP_K), lambda j, t, m: (0, 0)),
        ],
        out_specs=pl.BlockSpec((SEQ, D_MODEL), lambda j, t, m: (0, 0)),
        scratch_shapes=[
            pltpu.VMEM((NP_ROWS, D_MODEL), jnp.bfloat16),
            pltpu.VMEM((NP_ROWS, D_MODEL), jnp.float32),
            pltpu.VMEM((SEQ, D_MODEL), jnp.float32),
        ],
    )
    return pl.pallas_call(
        _moe_body,
        grid_spec=grid_spec,
        out_shape=jax.ShapeDtypeStruct((SEQ, D_MODEL), jnp.float32),
    )(
        meta,
        xb,
        W1,
        b1.reshape(N_EXPERTS, 1, D_FF),
        W2,
        b2.reshape(N_EXPERTS, 1, D_MODEL),
        dest2,
        wts,
    )


def kernel(x, Wg, bg, W1, b1, W2, b2):
    B, S, D = x.shape
    xt = x.reshape(S, D)

    wts, dest, meta2 = _gate_route(xt, Wg, bg)
    meta = meta2.reshape(2 * NT)
    dest2 = dest.reshape(TOP_K, SEQ)

    out = _moe(meta, xt.astype(jnp.bfloat16), W1, b1, W2, b2, dest2, wts)
    return out.reshape(B, S, D)


# speedup vs baseline: 2.6217x; 1.4619x over previous
import jax
import jax.numpy as jnp
from jax import lax
from jax.experimental import pallas as pl
from jax.experimental.pallas import tpu as pltpu

SEQ = 2048
D_MODEL = 768
N_EXPERTS = 8
TOP_K = 2
D_FF = 4 * D_MODEL

NPAIR = SEQ * TOP_K
TM = 256
NP_ROWS = NPAIR + N_EXPERTS * TM
NT = NP_ROWS // TM


def _gate_route_body(x_ref, wg_ref, bg_ref, wts_ref, dest_ref, meta_ref,
                     xb_ref):
    x = x_ref[...]
    xb_ref[...] = x.astype(jnp.bfloat16)
    logits = jnp.dot(x, wg_ref[...], preferred_element_type=jnp.float32)
    logits = logits + bg_ref[...]
    iota = lax.broadcasted_iota(jnp.int32, logits.shape, 1)
    m1 = jnp.max(logits, axis=1, keepdims=True)
    i1 = jnp.min(jnp.where(logits >= m1, iota, N_EXPERTS), axis=1, keepdims=True)
    lmask = jnp.where(iota == i1, -jnp.inf, logits)
    m2 = jnp.max(lmask, axis=1, keepdims=True)
    i2 = jnp.min(jnp.where(lmask >= m2, iota, N_EXPERTS), axis=1, keepdims=True)
    z = jnp.sum(jnp.exp(logits - m1), axis=1, keepdims=True)
    wts_ref[...] = jnp.concatenate([1.0 / z, jnp.exp(m2 - m1) / z], axis=1)

    iotaE1 = lax.broadcasted_iota(jnp.int32, (SEQ, N_EXPERTS), 1)
    oh = jnp.concatenate([(iotaE1 == i1), (iotaE1 == i2)], axis=0)
    oh = oh.astype(jnp.int32)
    s = oh
    d = 1
    while d < NPAIR:
        s = s + jnp.concatenate(
            [jnp.zeros((d, N_EXPERTS), jnp.int32), s[: NPAIR - d, :]], axis=0)
        d *= 2
    rank = jnp.sum(oh * s, axis=1, keepdims=True)
    counts = s[NPAIR - 1 :, :]
    cpad = ((counts + TM - 1) // TM) * TM
    e = cpad
    d = 1
    while d < N_EXPERTS:
        e = e + jnp.concatenate(
            [jnp.zeros((1, d), jnp.int32), e[:, : N_EXPERTS - d]], axis=1)
        d *= 2
    ends = e
    starts = ends - cpad
    start_pp = jnp.sum(oh * starts, axis=1, keepdims=True)
    dest_ref[...] = start_pp + rank - 1

    tvec = lax.broadcasted_iota(jnp.int32, (NT, N_EXPERTS), 0) * TM
    te = jnp.minimum(jnp.sum((tvec >= ends).astype(jnp.int32), axis=1,
                             keepdims=True), N_EXPERTS - 1)
    valid = (tvec[:, :1] < ends[:, N_EXPERTS - 1 :]).astype(jnp.int32)
    meta_ref[...] = jnp.concatenate([te, valid], axis=0)


def _gate_route(xt, Wg, bg):
    return pl.pallas_call(
        _gate_route_body,
        out_shape=(
            jax.ShapeDtypeStruct((SEQ, TOP_K), jnp.float32),
            jax.ShapeDtypeStruct((NPAIR, 1), jnp.int32),
            jax.ShapeDtypeStruct((2 * NT, 1), jnp.int32),
            jax.ShapeDtypeStruct((SEQ, D_MODEL), jnp.bfloat16),
        ),
    )(xt, Wg, bg.reshape(1, N_EXPERTS))


def _moe_body(meta_ref, x_ref, w1_ref, b1_ref, w2_ref, b2_ref, d2_ref,
              wts_ref, out_ref):
    t = pl.program_id(0)
    base = t * TM
    valid = meta_ref[NT + t] == 1

    @pl.when(valid)
    def _():
        d0r = d2_ref[0, :]
        d1r = d2_ref[1, :]
        iota_r = lax.broadcasted_iota(jnp.int32, (TM, SEQ), 0) + base
        hit = jnp.logical_or(d0r[None, :] == iota_r, d1r[None, :] == iota_r)
        p = jnp.where(hit, 1.0, 0.0).astype(jnp.bfloat16)
        xs = jnp.dot(p, x_ref[...], preferred_element_type=jnp.float32)
        xs = xs.astype(jnp.bfloat16)
        w1 = w1_ref[0].astype(jnp.bfloat16)
        h = jnp.dot(xs, w1, preferred_element_type=jnp.float32)
        h = jnp.maximum(h + b1_ref[0, 0][None, :], 0.0).astype(jnp.bfloat16)
        w2 = w2_ref[0].astype(jnp.bfloat16)
        y = jnp.dot(h, w2, preferred_element_type=jnp.float32)
        y = y + b2_ref[0, 0][None, :]
        w = wts_ref[...]
        iota_c = lax.broadcasted_iota(jnp.int32, (SEQ, TM), 1) + base
        gt = (jnp.where(d0r[:, None] == iota_c, w[:, :1], 0.0)
              + jnp.where(d1r[:, None] == iota_c, w[:, 1:2], 0.0)
              ).astype(jnp.bfloat16)
        contrib = jnp.dot(gt, y.astype(jnp.bfloat16),
                          preferred_element_type=jnp.float32)

        @pl.when(t == 0)
        def _():
            out_ref[...] = contrib

        @pl.when(t > 0)
        def _():
            out_ref[...] += contrib


def _moe(meta, xb, W1, b1, W2, b2, dest2, wts):
    grid_spec = pltpu.PrefetchScalarGridSpec(
        num_scalar_prefetch=1,
        grid=(NT,),
        in_specs=[
            pl.BlockSpec((SEQ, D_MODEL), lambda t, m: (0, 0)),
            pl.BlockSpec((1, D_MODEL, D_FF), lambda t, m: (m[t], 0, 0)),
            pl.BlockSpec((1, 1, D_FF), lambda t, m: (m[t], 0, 0)),
            pl.BlockSpec((1, D_FF, D_MODEL), lambda t, m: (m[t], 0, 0)),
            pl.BlockSpec((1, 1, D_MODEL), lambda t, m: (m[t], 0, 0)),
            pl.BlockSpec((2, SEQ), lambda t, m: (0, 0)),
            pl.BlockSpec((SEQ, TOP_K), lambda t, m: (0, 0)),
        ],
        out_specs=pl.BlockSpec((SEQ, D_MODEL), lambda t, m: (0, 0)),
    )
    return pl.pallas_call(
        _moe_body,
        grid_spec=grid_spec,
        out_shape=jax.ShapeDtypeStruct((SEQ, D_MODEL), jnp.float32),
    )(
        meta,
        xb,
        W1,
        b1.reshape(N_EXPERTS, 1, D_FF),
        W2,
        b2.reshape(N_EXPERTS, 1, D_MODEL),
        dest2,
        wts,
    )


def kernel(x, Wg, bg, W1, b1, W2, b2):
    B, S, D = x.shape
    xt = x.reshape(S, D)

    wts, dest, meta2, xb = _gate_route(xt, Wg, bg)
    meta = meta2.reshape(2 * NT)
    dest2 = dest.reshape(TOP_K, SEQ)

    out = _moe(meta, xb, W1, b1, W2, b2, dest2, wts)
    return out.reshape(B, S, D)
